# baseline (speedup 1.0000x reference)
import functools

import jax
import jax.numpy as jnp
from jax import lax
from jax.experimental import pallas as pl
from jax.experimental.pallas import tpu as pltpu
from jax.experimental.pallas import tpu_sc as plsc

N = 10000
E = 320000
D = 128
NC = 2
NS = 16
NW = NC * NS
EPT = E // NW
K = 80
NCHUNK = EPT // K
RPT = N // NS
ZR = 125
DEGW = 16
L = 16

_mesh = plsc.VectorSubcoreMesh(core_axis_name="c", subcore_axis_name="s")


def _fill(ref, rows, width, value):
    def body(i, _):
        r = i // (width // L)
        j = i % (width // L)
        ref[r, pl.ds(j * L, L)] = jnp.full((L,), value, jnp.float32)
        return 0

    lax.fori_loop(0, rows * (width // L), body, 0)


PROW = N // L


def _deg_body(dst_hbm, out_hbm, dv, pdeg):
    c = lax.axis_index("c")
    s = lax.axis_index("s")
    wid = c * NS + s
    pltpu.sync_copy(dst_hbm.at[pl.ds(wid * EPT, EPT)], dv)

    def z(i, _):
        pdeg[pl.ds(i * L, L)] = jnp.zeros((L,), jnp.float32)
        return 0

    lax.fori_loop(0, N // L, z, 0)
    ones = jnp.ones((L,), jnp.float32)

    def acc(i, _):
        idx = dv[pl.ds(i * L, L)]
        plsc.addupdate_scatter(pdeg, [idx], ones)
        return 0

    lax.fori_loop(0, EPT // L, acc, 0)
    pltpu.sync_copy(pdeg, out_hbm.at[c, s])


_deg_call = functools.partial(
    pl.kernel,
    out_type=jax.ShapeDtypeStruct((NC, NS, N), jnp.float32),
    mesh=_mesh,
    compiler_params=pltpu.CompilerParams(needs_layout_passes=False),
    scratch_types=[
        pltpu.VMEM((EPT,), jnp.int32),
        pltpu.VMEM((N,), jnp.float32),
    ],
)(_deg_body)


NB = 3


def _agg_body(p_hbm, pk_hbm, out_hbm, pks, sidx3, didx3, buf0, buf1, buf2,
              acc_sh, g0, g1, g2, i0, i1, i2):
    c = lax.axis_index("c")
    s = lax.axis_index("s")
    wid = c * NS + s
    ebase = wid * EPT
    bufs = [buf0, buf1, buf2]
    gsems = [g0, g1, g2]
    isems = [i0, i1, i2]

    def fire_idx(q, j):
        pltpu.async_copy(pk_hbm.at[pl.ds(ebase + q * K, K)], pks.at[j],
                         isems[j])

    def wait_idx(j):
        pltpu.make_async_copy(pk_hbm.at[pl.ds(ebase, K)], pks.at[j],
                              isems[j]).wait()

    def unpack(j):
        for t in range(K // L):
            v = pks[j, pl.ds(t * L, L)]
            sidx3[j, pl.ds(t * L, L)] = lax.shift_right_logical(v, 14)
            didx3[j, pl.ds(t * L, L)] = lax.bitwise_and(v, 16383)

    def fire_g(j):
        pltpu.async_copy(p_hbm.at[sidx3.at[j]], bufs[j], gsems[j])

    def wait_g(j):
        pltpu.make_async_copy(p_hbm.at[sidx3.at[j]], bufs[j],
                              gsems[j]).wait()

    def scat(j):
        pltpu.sync_copy(bufs[j], acc_sh.at[didx3.at[j]], add=True)

    fire_idx(0, 0)
    fire_idx(1, 1)
    fire_idx(2, 2)

    _fill(buf0, K, D, 0.0)
    row0 = s * RPT
    for t in range(RPT // K):
        pltpu.sync_copy(buf0, acc_sh.at[pl.ds(row0 + t * K, K)])
    rem = RPT % K
    if rem:
        pltpu.sync_copy(buf0.at[pl.ds(0, rem)],
                        acc_sh.at[pl.ds(row0 + (RPT // K) * K, rem)])
    plsc.subcore_barrier()

    wait_idx(0)
    unpack(0)
    fire_g(0)
    wait_idx(1)
    unpack(1)
    fire_g(1)

    def step(i, _):
        for jj in range(NB):
            q = NB * i + jj
            jn = (jj + 2) % NB
            wait_idx(jn)
            unpack(jn)
            fire_g(jn)
            wait_g(jj)
            scat(jj)

            @pl.when(q + NB <= NCHUNK - 1)
            def _():
                fire_idx(q + NB, jj)

        return 0

    lax.fori_loop(0, (NCHUNK - 2) // NB, step, 0)
    wait_g((NCHUNK - 2) % NB)
    scat((NCHUNK - 2) % NB)
    wait_g((NCHUNK - 1) % NB)
    scat((NCHUNK - 1) % NB)
    plsc.subcore_barrier()
    pltpu.sync_copy(acc_sh.at[pl.ds(row0, RPT)], out_hbm.at[c, s])


_agg_call = functools.partial(
    pl.kernel,
    out_type=jax.ShapeDtypeStruct((NC, NS, RPT, D), jnp.float32),
    mesh=_mesh,
    scratch_types=[
        pltpu.VMEM((NB, K), jnp.int32),
        pltpu.VMEM((NB, K), jnp.int32),
        pltpu.VMEM((NB, K), jnp.int32),
        pltpu.VMEM((K, D), jnp.float32),
        pltpu.VMEM((K, D), jnp.float32),
        pltpu.VMEM((K, D), jnp.float32),
        pltpu.VMEM_SHARED((N, D), jnp.float32),
        pltpu.SemaphoreType.DMA,
        pltpu.SemaphoreType.DMA,
        pltpu.SemaphoreType.DMA,
        pltpu.SemaphoreType.DMA,
        pltpu.SemaphoreType.DMA,
        pltpu.SemaphoreType.DMA,
    ],
)(_agg_body)


BM = 1000


def _tc1_body(x_ref, w_ref, degp_ref, p_ref, dinv_ref):
    deg = jnp.sum(degp_ref[...], axis=1, keepdims=True) + 1.0
    dcol = lax.rsqrt(deg)
    dinv_ref[...] = jnp.broadcast_to(dcol, (BM, DEGW))
    p_ref[...] = jnp.dot(x_ref[...], w_ref[...],
                         preferred_element_type=jnp.float32) * dcol


_tc1_call = pl.pallas_call(
    _tc1_body,
    grid=(N // BM,),
    in_specs=[
        pl.BlockSpec((BM, D), lambda i: (i, 0)),
        pl.BlockSpec((D, D), lambda i: (0, 0)),
        pl.BlockSpec((BM, NW), lambda i: (i, 0)),
    ],
    out_specs=[
        pl.BlockSpec((BM, D), lambda i: (i, 0)),
        pl.BlockSpec((BM, DEGW), lambda i: (i, 0)),
    ],
    out_shape=[
        jax.ShapeDtypeStruct((N, D), jnp.float32),
        jax.ShapeDtypeStruct((N, DEGW), jnp.float32),
    ],
)


def _tcmid_body(sp_ref, p_ref, dinv_ref, b_ref, w_ref, out_ref):
    dcol = dinv_ref[:, 0:1]
    t = (sp_ref[0] + sp_ref[1] + p_ref[...]) * dcol + b_ref[...]
    t = jnp.maximum(t, 0.0)
    out_ref[...] = jnp.dot(t, w_ref[...],
                           preferred_element_type=jnp.float32) * dcol


_tcmid_call = pl.pallas_call(
    _tcmid_body,
    grid=(N // BM,),
    in_specs=[
        pl.BlockSpec((NC, BM, D), lambda i: (0, i, 0)),
        pl.BlockSpec((BM, D), lambda i: (i, 0)),
        pl.BlockSpec((BM, DEGW), lambda i: (i, 0)),
        pl.BlockSpec((1, D), lambda i: (0, 0)),
        pl.BlockSpec((D, D), lambda i: (0, 0)),
    ],
    out_specs=pl.BlockSpec((BM, D), lambda i: (i, 0)),
    out_shape=jax.ShapeDtypeStruct((N, D), jnp.float32),
)


def _tcfin_body(sp_ref, p_ref, dinv_ref, b_ref, out_ref):
    dcol = dinv_ref[:, 0:1]
    out_ref[...] = (sp_ref[0] + sp_ref[1] + p_ref[...]) * dcol + b_ref[...]


_tcfin_call = pl.pallas_call(
    _tcfin_body,
    grid=(N // BM,),
    in_specs=[
        pl.BlockSpec((NC, BM, D), lambda i: (0, i, 0)),
        pl.BlockSpec((BM, D), lambda i: (i, 0)),
        pl.BlockSpec((BM, DEGW), lambda i: (i, 0)),
        pl.BlockSpec((1, D), lambda i: (0, 0)),
    ],
    out_specs=pl.BlockSpec((BM, D), lambda i: (i, 0)),
    out_shape=jax.ShapeDtypeStruct((N, D), jnp.float32),
)


def kernel(x, edge_index, W1, b1, W2, b2, W3, b3):
    dst = edge_index[1]
    pk1 = (edge_index[0] << 14) | edge_index[1]
    degp = _deg_call(dst).reshape(NW, N).T
    p1, dinv16 = _tc1_call(x, W1, degp)
    sp1 = _agg_call(p1, pk1).reshape(NC, N, D)
    p2 = _tcmid_call(sp1, p1, dinv16, b1.reshape(1, D), W2)
    sp2 = _agg_call(p2, pk1).reshape(NC, N, D)
    p3 = _tcmid_call(sp2, p2, dinv16, b2.reshape(1, D), W3)
    sp3 = _agg_call(p3, pk1).reshape(NC, N, D)
    return _tcfin_call(sp3, p3, dinv16, b3.reshape(1, D))


# speedup vs baseline: 1.0815x; 1.0815x over previous
import functools

import jax
import jax.numpy as jnp
from jax import lax
from jax.experimental import pallas as pl
from jax.experimental.pallas import tpu as pltpu
from jax.experimental.pallas import tpu_sc as plsc

N = 10000
E = 320000
D = 128
NC = 2
NS = 16
NW = NC * NS
EPT = E // NW
K = 80
NCHUNK = EPT // K
RPT = N // NS
ZR = 125
DEGW = 16
L = 16

_mesh = plsc.VectorSubcoreMesh(core_axis_name="c", subcore_axis_name="s")


def _fill(ref, rows, width, value):
    def body(i, _):
        r = i // (width // L)
        j = i % (width // L)
        ref[r, pl.ds(j * L, L)] = jnp.full((L,), value, jnp.float32)
        return 0

    lax.fori_loop(0, rows * (width // L), body, 0)


PROW = N // L


def _deg_body(dst_hbm, out_hbm, dv, pdeg):
    c = lax.axis_index("c")
    s = lax.axis_index("s")
    wid = c * NS + s
    pltpu.sync_copy(dst_hbm.at[pl.ds(wid * EPT, EPT)], dv)

    def z(i, _):
        pdeg[pl.ds(i * L, L)] = jnp.zeros((L,), jnp.float32)
        return 0

    lax.fori_loop(0, N // L, z, 0)
    ones = jnp.ones((L,), jnp.float32)

    def acc(i, _):
        idx = dv[pl.ds(i * L, L)]
        plsc.addupdate_scatter(pdeg, [idx], ones)
        return 0

    lax.fori_loop(0, EPT // L, acc, 0)
    pltpu.sync_copy(pdeg, out_hbm.at[c, s])


_deg_call = functools.partial(
    pl.kernel,
    out_type=jax.ShapeDtypeStruct((NC, NS, N), jnp.float32),
    mesh=_mesh,
    compiler_params=pltpu.CompilerParams(needs_layout_passes=False),
    scratch_types=[
        pltpu.VMEM((EPT,), jnp.int32),
        pltpu.VMEM((N,), jnp.float32),
    ],
)(_deg_body)


def _agg_body(p_hbm, pk_hbm, out_hbm, pk, sidx2, didx2, buf0, buf1,
              acc_sh, gs0, gs1):
    c = lax.axis_index("c")
    s = lax.axis_index("s")
    wid = c * NS + s
    pltpu.sync_copy(pk_hbm.at[pl.ds(wid * EPT, EPT)], pk)

    def unpack(i, slot):
        for j in range(K // L):
            v = pk[pl.ds(i * K + j * L, L)]
            sidx2[slot, pl.ds(j * L, L)] = lax.shift_right_logical(v, 14)
            didx2[slot, pl.ds(j * L, L)] = lax.bitwise_and(v, 16383)

    _fill(buf0, K, D, 0.0)
    row0 = s * RPT
    for t in range(RPT // K):
        pltpu.sync_copy(buf0, acc_sh.at[pl.ds(row0 + t * K, K)])
    rem = RPT % K
    if rem:
        pltpu.sync_copy(buf0.at[pl.ds(0, rem)],
                        acc_sh.at[pl.ds(row0 + (RPT // K) * K, rem)])
    plsc.subcore_barrier()

    unpack(0, 0)
    pltpu.async_copy(p_hbm.at[sidx2.at[0]], buf0, gs0)

    def pair(i2, _):
        a = 2 * i2
        unpack(a + 1, 1)
        pltpu.async_copy(p_hbm.at[sidx2.at[1]], buf1, gs1)
        pltpu.make_async_copy(p_hbm.at[sidx2.at[0]], buf0, gs0).wait()
        pltpu.sync_copy(buf0, acc_sh.at[didx2.at[0]], add=True)
        unpack(a + 2, 0)
        pltpu.async_copy(p_hbm.at[sidx2.at[0]], buf0, gs0)
        pltpu.make_async_copy(p_hbm.at[sidx2.at[1]], buf1, gs1).wait()
        pltpu.sync_copy(buf1, acc_sh.at[didx2.at[1]], add=True)
        return 0

    lax.fori_loop(0, (NCHUNK - 1) // 2, pair, 0)
    pltpu.make_async_copy(p_hbm.at[sidx2.at[0]], buf0, gs0).wait()
    pltpu.sync_copy(buf0, acc_sh.at[didx2.at[0]], add=True)
    plsc.subcore_barrier()
    pltpu.sync_copy(acc_sh.at[pl.ds(row0, RPT)], out_hbm.at[c, s])


_agg_call = functools.partial(
    pl.kernel,
    out_type=jax.ShapeDtypeStruct((NC, NS, RPT, D), jnp.float32),
    mesh=_mesh,
    scratch_types=[
        pltpu.VMEM((EPT,), jnp.int32),
        pltpu.VMEM((2, K), jnp.int32),
        pltpu.VMEM((2, K), jnp.int32),
        pltpu.VMEM((K, D), jnp.float32),
        pltpu.VMEM((K, D), jnp.float32),
        pltpu.VMEM_SHARED((N, D), jnp.float32),
        pltpu.SemaphoreType.DMA,
        pltpu.SemaphoreType.DMA,
    ],
)(_agg_body)


BM = 1000


def _tc1_body(x_ref, w_ref, degp_ref, p_ref, dinv_ref):
    deg = jnp.sum(degp_ref[...], axis=1, keepdims=True) + 1.0
    dcol = lax.rsqrt(deg)
    dinv_ref[...] = jnp.broadcast_to(dcol, (BM, DEGW))
    p_ref[...] = jnp.dot(x_ref[...], w_ref[...],
                         preferred_element_type=jnp.float32) * dcol


_tc1_call = pl.pallas_call(
    _tc1_body,
    grid=(N // BM,),
    in_specs=[
        pl.BlockSpec((BM, D), lambda i: (i, 0)),
        pl.BlockSpec((D, D), lambda i: (0, 0)),
        pl.BlockSpec((BM, NW), lambda i: (i, 0)),
    ],
    out_specs=[
        pl.BlockSpec((BM, D), lambda i: (i, 0)),
        pl.BlockSpec((BM, DEGW), lambda i: (i, 0)),
    ],
    out_shape=[
        jax.ShapeDtypeStruct((N, D), jnp.float32),
        jax.ShapeDtypeStruct((N, DEGW), jnp.float32),
    ],
)


def _tcmid_body(sp_ref, p_ref, dinv_ref, b_ref, w_ref, out_ref):
    dcol = dinv_ref[:, 0:1]
    t = (sp_ref[0] + sp_ref[1] + p_ref[...]) * dcol + b_ref[...]
    t = jnp.maximum(t, 0.0)
    out_ref[...] = jnp.dot(t, w_ref[...],
                           preferred_element_type=jnp.float32) * dcol


_tcmid_call = pl.pallas_call(
    _tcmid_body,
    grid=(N // BM,),
    in_specs=[
        pl.BlockSpec((NC, BM, D), lambda i: (0, i, 0)),
        pl.BlockSpec((BM, D), lambda i: (i, 0)),
        pl.BlockSpec((BM, DEGW), lambda i: (i, 0)),
        pl.BlockSpec((1, D), lambda i: (0, 0)),
        pl.BlockSpec((D, D), lambda i: (0, 0)),
    ],
    out_specs=pl.BlockSpec((BM, D), lambda i: (i, 0)),
    out_shape=jax.ShapeDtypeStruct((N, D), jnp.float32),
)


def _tcfin_body(sp_ref, p_ref, dinv_ref, b_ref, out_ref):
    dcol = dinv_ref[:, 0:1]
    out_ref[...] = (sp_ref[0] + sp_ref[1] + p_ref[...]) * dcol + b_ref[...]


_tcfin_call = pl.pallas_call(
    _tcfin_body,
    grid=(N // BM,),
    in_specs=[
        pl.BlockSpec((NC, BM, D), lambda i: (0, i, 0)),
        pl.BlockSpec((BM, D), lambda i: (i, 0)),
        pl.BlockSpec((BM, DEGW), lambda i: (i, 0)),
        pl.BlockSpec((1, D), lambda i: (0, 0)),
    ],
    out_specs=pl.BlockSpec((BM, D), lambda i: (i, 0)),
    out_shape=jax.ShapeDtypeStruct((N, D), jnp.float32),
)


def kernel(x, edge_index, W1, b1, W2, b2, W3, b3):
    dst = edge_index[1]
    pk1 = (edge_index[0] << 14) | edge_index[1]
    degp = _deg_call(dst).reshape(NW, N).T
    p1, dinv16 = _tc1_call(x, W1, degp)
    sp1 = _agg_call(p1, pk1).reshape(NC, N, D)
    p2 = _tcmid_call(sp1, p1, dinv16, b1.reshape(1, D), W2)
    sp2 = _agg_call(p2, pk1).reshape(NC, N, D)
    p3 = _tcmid_call(sp2, p2, dinv16, b2.reshape(1, D), W3)
    sp3 = _agg_call(p3, pk1).reshape(NC, N, D)
    return _tcfin_call(sp3, p3, dinv16, b3.reshape(1, D))


# speedup vs baseline: 1.2468x; 1.1529x over previous
import functools

import jax
import jax.numpy as jnp
from jax import lax
from jax.experimental import pallas as pl
from jax.experimental.pallas import tpu as pltpu
from jax.experimental.pallas import tpu_sc as plsc

N = 10000
E = 320000
D = 128
NC = 2
NS = 16
NW = NC * NS
EPT = E // NW
K = 80
NCHUNK = EPT // K
RPT = N // NS
ZR = 125
DEGW = 16
L = 16

_mesh = plsc.VectorSubcoreMesh(core_axis_name="c", subcore_axis_name="s")


def _fill(ref, rows, width, value):
    def body(i, _):
        r = i // (width // L)
        j = i % (width // L)
        ref[r, pl.ds(j * L, L)] = jnp.full((L,), value, jnp.float32)
        return 0

    lax.fori_loop(0, rows * (width // L), body, 0)


PROW = N // L


def _deg_body(dst_hbm, out_hbm, dv, pdeg):
    c = lax.axis_index("c")
    s = lax.axis_index("s")
    wid = c * NS + s
    pltpu.sync_copy(dst_hbm.at[pl.ds(wid * EPT, EPT)], dv)

    def z(i, _):
        pdeg[pl.ds(i * L, L)] = jnp.zeros((L,), jnp.float32)
        return 0

    lax.fori_loop(0, N // L, z, 0)
    ones = jnp.ones((L,), jnp.float32)

    def acc(i, _):
        idx = dv[pl.ds(i * L, L)]
        plsc.addupdate_scatter(pdeg, [idx], ones)
        return 0

    lax.fori_loop(0, EPT // L, acc, 0)
    pltpu.sync_copy(pdeg, out_hbm.at[c, s])


_deg_call = functools.partial(
    pl.kernel,
    out_type=jax.ShapeDtypeStruct((NC, NS, N), jnp.float32),
    mesh=_mesh,
    compiler_params=pltpu.CompilerParams(needs_layout_passes=False),
    scratch_types=[
        pltpu.VMEM((EPT,), jnp.int32),
        pltpu.VMEM((N,), jnp.float32),
    ],
)(_deg_body)


NB = 3


def _agg_body(p_hbm, pk_hbm, out_hbm, pks, sidx3, didx3, buf0, buf1, buf2,
              acc_sh, g0, g1, g2, i0, i1, i2, s0, s1, s2):
    c = lax.axis_index("c")
    s = lax.axis_index("s")
    wid = c * NS + s
    ebase = wid * EPT
    bufs = [buf0, buf1, buf2]
    gsems = [g0, g1, g2]
    isems = [i0, i1, i2]
    ssems = [s0, s1, s2]

    def fire_idx(q, j):
        pltpu.async_copy(pk_hbm.at[pl.ds(ebase + q * K, K)], pks.at[j],
                         isems[j])

    def wait_idx(j):
        pltpu.make_async_copy(pk_hbm.at[pl.ds(ebase, K)], pks.at[j],
                              isems[j]).wait()

    def unpack(j):
        for t in range(K // L):
            v = pks[j, pl.ds(t * L, L)]
            sidx3[j, pl.ds(t * L, L)] = lax.shift_right_logical(v, 14)
            didx3[j, pl.ds(t * L, L)] = lax.bitwise_and(v, 16383)

    def fire_g(j):
        pltpu.async_copy(p_hbm.at[sidx3.at[j]], bufs[j], gsems[j])

    def wait_g(j):
        pltpu.make_async_copy(p_hbm.at[sidx3.at[j]], bufs[j],
                              gsems[j]).wait()

    def fire_scat(j):
        pltpu.async_copy(bufs[j], acc_sh.at[didx3.at[j]], ssems[j],
                         add=True)

    def wait_scat(j):
        pltpu.make_async_copy(bufs[j], acc_sh.at[didx3.at[j]],
                              ssems[j]).wait()

    fire_idx(0, 0)
    fire_idx(1, 1)
    fire_idx(2, 2)

    _fill(buf0, K, D, 0.0)
    row0 = s * RPT
    for t in range(RPT // K):
        pltpu.sync_copy(buf0, acc_sh.at[pl.ds(row0 + t * K, K)])
    rem = RPT % K
    if rem:
        pltpu.sync_copy(buf0.at[pl.ds(0, rem)],
                        acc_sh.at[pl.ds(row0 + (RPT // K) * K, rem)])
    plsc.subcore_barrier()

    wait_idx(0)
    unpack(0)
    fire_g(0)
    wait_idx(1)
    unpack(1)
    fire_g(1)

    def step(i, _):
        for jj in range(NB):
            q = NB * i + jj
            jn = (jj + 2) % NB
            wait_idx(jn)
            if jj == 0:
                @pl.when(i > 0)
                def _():
                    wait_scat(jn)
            else:
                wait_scat(jn)
            unpack(jn)
            fire_g(jn)
            wait_g(jj)
            fire_scat(jj)

            @pl.when(q + NB <= NCHUNK - 1)
            def _():
                fire_idx(q + NB, jj)

        return 0

    lax.fori_loop(0, (NCHUNK - 2) // NB, step, 0)
    wait_g((NCHUNK - 2) % NB)
    fire_scat((NCHUNK - 2) % NB)
    wait_g((NCHUNK - 1) % NB)
    fire_scat((NCHUNK - 1) % NB)
    for j in range(NB):
        wait_scat(j)
    plsc.subcore_barrier()
    pltpu.sync_copy(acc_sh.at[pl.ds(row0, RPT)], out_hbm.at[c, s])


_agg_call = functools.partial(
    pl.kernel,
    out_type=jax.ShapeDtypeStruct((NC, NS, RPT, D), jnp.float32),
    mesh=_mesh,
    scratch_types=[
        pltpu.VMEM((NB, K), jnp.int32),
        pltpu.VMEM((NB, K), jnp.int32),
        pltpu.VMEM((NB, K), jnp.int32),
        pltpu.VMEM((K, D), jnp.float32),
        pltpu.VMEM((K, D), jnp.float32),
        pltpu.VMEM((K, D), jnp.float32),
        pltpu.VMEM_SHARED((N, D), jnp.float32),
        pltpu.SemaphoreType.DMA,
        pltpu.SemaphoreType.DMA,
        pltpu.SemaphoreType.DMA,
        pltpu.SemaphoreType.DMA,
        pltpu.SemaphoreType.DMA,
        pltpu.SemaphoreType.DMA,
        pltpu.SemaphoreType.DMA,
        pltpu.SemaphoreType.DMA,
        pltpu.SemaphoreType.DMA,
    ],
)(_agg_body)


BM = 1000


def _tc1_body(x_ref, w_ref, degp_ref, p_ref, dinv_ref):
    deg = jnp.sum(degp_ref[...], axis=1, keepdims=True) + 1.0
    dcol = lax.rsqrt(deg)
    dinv_ref[...] = jnp.broadcast_to(dcol, (BM, DEGW))
    p_ref[...] = jnp.dot(x_ref[...], w_ref[...],
                         preferred_element_type=jnp.float32) * dcol


_tc1_call = pl.pallas_call(
    _tc1_body,
    grid=(N // BM,),
    in_specs=[
        pl.BlockSpec((BM, D), lambda i: (i, 0)),
        pl.BlockSpec((D, D), lambda i: (0, 0)),
        pl.BlockSpec((BM, NW), lambda i: (i, 0)),
    ],
    out_specs=[
        pl.BlockSpec((BM, D), lambda i: (i, 0)),
        pl.BlockSpec((BM, DEGW), lambda i: (i, 0)),
    ],
    out_shape=[
        jax.ShapeDtypeStruct((N, D), jnp.float32),
        jax.ShapeDtypeStruct((N, DEGW), jnp.float32),
    ],
)


def _tcmid_body(sp_ref, p_ref, dinv_ref, b_ref, w_ref, out_ref):
    dcol = dinv_ref[:, 0:1]
    t = (sp_ref[0] + sp_ref[1] + p_ref[...]) * dcol + b_ref[...]
    t = jnp.maximum(t, 0.0)
    out_ref[...] = jnp.dot(t, w_ref[...],
                           preferred_element_type=jnp.float32) * dcol


_tcmid_call = pl.pallas_call(
    _tcmid_body,
    grid=(N // BM,),
    in_specs=[
        pl.BlockSpec((NC, BM, D), lambda i: (0, i, 0)),
        pl.BlockSpec((BM, D), lambda i: (i, 0)),
        pl.BlockSpec((BM, DEGW), lambda i: (i, 0)),
        pl.BlockSpec((1, D), lambda i: (0, 0)),
        pl.BlockSpec((D, D), lambda i: (0, 0)),
    ],
    out_specs=pl.BlockSpec((BM, D), lambda i: (i, 0)),
    out_shape=jax.ShapeDtypeStruct((N, D), jnp.float32),
)


def _tcfin_body(sp_ref, p_ref, dinv_ref, b_ref, out_ref):
    dcol = dinv_ref[:, 0:1]
    out_ref[...] = (sp_ref[0] + sp_ref[1] + p_ref[...]) * dcol + b_ref[...]


_tcfin_call = pl.pallas_call(
    _tcfin_body,
    grid=(N // BM,),
    in_specs=[
        pl.BlockSpec((NC, BM, D), lambda i: (0, i, 0)),
        pl.BlockSpec((BM, D), lambda i: (i, 0)),
        pl.BlockSpec((BM, DEGW), lambda i: (i, 0)),
        pl.BlockSpec((1, D), lambda i: (0, 0)),
    ],
    out_specs=pl.BlockSpec((BM, D), lambda i: (i, 0)),
    out_shape=jax.ShapeDtypeStruct((N, D), jnp.float32),
)


def kernel(x, edge_index, W1, b1, W2, b2, W3, b3):
    dst = edge_index[1]
    pk1 = (edge_index[0] << 14) | edge_index[1]
    degp = _deg_call(dst).reshape(NW, N).T
    p1, dinv16 = _tc1_call(x, W1, degp)
    sp1 = _agg_call(p1, pk1).reshape(NC, N, D)
    p2 = _tcmid_call(sp1, p1, dinv16, b1.reshape(1, D), W2)
    sp2 = _agg_call(p2, pk1).reshape(NC, N, D)
    p3 = _tcmid_call(sp2, p2, dinv16, b2.reshape(1, D), W3)
    sp3 = _agg_call(p3, pk1).reshape(NC, N, D)
    return _tcfin_call(sp3, p3, dinv16, b3.reshape(1, D))


# speedup vs baseline: 1.2728x; 1.0209x over previous
import functools

import jax
import jax.numpy as jnp
from jax import lax
from jax.experimental import pallas as pl
from jax.experimental.pallas import tpu as pltpu
from jax.experimental.pallas import tpu_sc as plsc

N = 10000
E = 320000
D = 128
NC = 2
NS = 16
NW = NC * NS
EPT = E // NW
K = 80
NCHUNK = EPT // K
RPT = N // NS
ZR = 125
DEGW = 16
L = 16

_mesh = plsc.VectorSubcoreMesh(core_axis_name="c", subcore_axis_name="s")


def _fill(ref, rows, width, value):
    def body(i, _):
        r = i // (width // L)
        j = i % (width // L)
        ref[r, pl.ds(j * L, L)] = jnp.full((L,), value, jnp.float32)
        return 0

    lax.fori_loop(0, rows * (width // L), body, 0)


PROW = N // L


def _deg_body(dst_hbm, out_hbm, dv, pdeg):
    c = lax.axis_index("c")
    s = lax.axis_index("s")
    wid = c * NS + s
    pltpu.sync_copy(dst_hbm.at[pl.ds(wid * EPT, EPT)], dv)

    def z(i, _):
        pdeg[pl.ds(i * L, L)] = jnp.zeros((L,), jnp.float32)
        return 0

    lax.fori_loop(0, N // L, z, 0)
    ones = jnp.ones((L,), jnp.float32)

    def acc(i, _):
        idx = dv[pl.ds(i * L, L)]
        plsc.addupdate_scatter(pdeg, [idx], ones)
        return 0

    lax.fori_loop(0, EPT // L, acc, 0)
    pltpu.sync_copy(pdeg, out_hbm.at[c, s])


_deg_call = functools.partial(
    pl.kernel,
    out_type=jax.ShapeDtypeStruct((NC, NS, N), jnp.float32),
    mesh=_mesh,
    compiler_params=pltpu.CompilerParams(needs_layout_passes=False),
    scratch_types=[
        pltpu.VMEM((EPT,), jnp.int32),
        pltpu.VMEM((N,), jnp.float32),
    ],
)(_deg_body)


NB = 3


def _agg_body(p_hbm, pk_hbm, out_hbm, pks, sidx3, didx3, buf0, buf1, buf2,
              acc_sh, g0, g1, g2, i0, i1, i2, s0, s1, s2):
    c = lax.axis_index("c")
    s = lax.axis_index("s")
    wid = c * NS + s
    ebase = wid * EPT
    bufs = [buf0, buf1, buf2]
    gsems = [g0, g1, g2]
    isems = [i0, i1, i2]
    ssems = [s0, s1, s2]

    def fire_idx(q, j):
        pltpu.async_copy(pk_hbm.at[pl.ds(ebase + q * K, K)], pks.at[j],
                         isems[j])

    def wait_idx(j):
        pltpu.make_async_copy(pk_hbm.at[pl.ds(ebase, K)], pks.at[j],
                              isems[j]).wait()

    def unpack_s(j):
        for t in range(K // L):
            v = pks[j, pl.ds(t * L, L)]
            sidx3[j, pl.ds(t * L, L)] = lax.shift_right_logical(v, 14)

    def unpack_d(j):
        for t in range(K // L):
            v = pks[j, pl.ds(t * L, L)]
            didx3[j, pl.ds(t * L, L)] = lax.bitwise_and(v, 16383)

    def fire_g(j):
        pltpu.async_copy(p_hbm.at[sidx3.at[j]], bufs[j], gsems[j])

    def wait_g(j):
        pltpu.make_async_copy(p_hbm.at[sidx3.at[j]], bufs[j],
                              gsems[j]).wait()

    def fire_scat(j):
        pltpu.async_copy(bufs[j], acc_sh.at[didx3.at[j]], ssems[j],
                         add=True)

    def wait_scat(j):
        pltpu.make_async_copy(bufs[j], acc_sh.at[didx3.at[j]],
                              ssems[j]).wait()

    fire_idx(0, 0)
    fire_idx(1, 1)
    fire_idx(2, 2)

    _fill(buf0, K, D, 0.0)
    row0 = s * RPT
    for t in range(RPT // K):
        pltpu.sync_copy(buf0, acc_sh.at[pl.ds(row0 + t * K, K)])
    rem = RPT % K
    if rem:
        pltpu.sync_copy(buf0.at[pl.ds(0, rem)],
                        acc_sh.at[pl.ds(row0 + (RPT // K) * K, rem)])
    plsc.subcore_barrier()

    wait_idx(0)
    unpack_s(0)
    unpack_d(0)
    fire_g(0)
    wait_idx(1)
    unpack_s(1)
    unpack_d(1)
    fire_g(1)

    def step(i, _):
        for jj in range(NB):
            q = NB * i + jj
            jn = (jj + 2) % NB
            wait_idx(jn)
            unpack_s(jn)
            if jj == 0:
                @pl.when(i > 0)
                def _():
                    wait_scat(jn)
            else:
                wait_scat(jn)
            unpack_d(jn)
            fire_g(jn)
            wait_g(jj)
            fire_scat(jj)

            @pl.when(q + NB <= NCHUNK - 1)
            def _():
                fire_idx(q + NB, jj)

        return 0

    lax.fori_loop(0, (NCHUNK - 2) // NB, step, 0)
    wait_g((NCHUNK - 2) % NB)
    fire_scat((NCHUNK - 2) % NB)
    wait_g((NCHUNK - 1) % NB)
    fire_scat((NCHUNK - 1) % NB)
    for j in range(NB):
        wait_scat(j)
    plsc.subcore_barrier()
    pltpu.sync_copy(acc_sh.at[pl.ds(row0, RPT)], out_hbm.at[c, s])


_agg_call = functools.partial(
    pl.kernel,
    out_type=jax.ShapeDtypeStruct((NC, NS, RPT, D), jnp.float32),
    mesh=_mesh,
    scratch_types=[
        pltpu.VMEM((NB, K), jnp.int32),
        pltpu.VMEM((NB, K), jnp.int32),
        pltpu.VMEM((NB, K), jnp.int32),
        pltpu.VMEM((K, D), jnp.float32),
        pltpu.VMEM((K, D), jnp.float32),
        pltpu.VMEM((K, D), jnp.float32),
        pltpu.VMEM_SHARED((N, D), jnp.float32),
        pltpu.SemaphoreType.DMA,
        pltpu.SemaphoreType.DMA,
        pltpu.SemaphoreType.DMA,
        pltpu.SemaphoreType.DMA,
        pltpu.SemaphoreType.DMA,
        pltpu.SemaphoreType.DMA,
        pltpu.SemaphoreType.DMA,
        pltpu.SemaphoreType.DMA,
        pltpu.SemaphoreType.DMA,
    ],
)(_agg_body)


BM = 2000


def _tc1_body(x_ref, w_ref, degp_ref, p_ref, dinv_ref):
    deg = jnp.sum(degp_ref[...], axis=1, keepdims=True) + 1.0
    dcol = lax.rsqrt(deg)
    dinv_ref[...] = jnp.broadcast_to(dcol, (BM, DEGW))
    p_ref[...] = jnp.dot(x_ref[...], w_ref[...],
                         preferred_element_type=jnp.float32) * dcol


_tc1_call = pl.pallas_call(
    _tc1_body,
    grid=(N // BM,),
    in_specs=[
        pl.BlockSpec((BM, D), lambda i: (i, 0)),
        pl.BlockSpec((D, D), lambda i: (0, 0)),
        pl.BlockSpec((BM, NW), lambda i: (i, 0)),
    ],
    out_specs=[
        pl.BlockSpec((BM, D), lambda i: (i, 0)),
        pl.BlockSpec((BM, DEGW), lambda i: (i, 0)),
    ],
    out_shape=[
        jax.ShapeDtypeStruct((N, D), jnp.float32),
        jax.ShapeDtypeStruct((N, DEGW), jnp.float32),
    ],
)


def _tcmid_body(sp_ref, p_ref, dinv_ref, b_ref, w_ref, out_ref):
    dcol = dinv_ref[:, 0:1]
    t = (sp_ref[0] + sp_ref[1] + p_ref[...]) * dcol + b_ref[...]
    t = jnp.maximum(t, 0.0)
    out_ref[...] = jnp.dot(t, w_ref[...],
                           preferred_element_type=jnp.float32) * dcol


_tcmid_call = pl.pallas_call(
    _tcmid_body,
    grid=(N // BM,),
    in_specs=[
        pl.BlockSpec((NC, BM, D), lambda i: (0, i, 0)),
        pl.BlockSpec((BM, D), lambda i: (i, 0)),
        pl.BlockSpec((BM, DEGW), lambda i: (i, 0)),
        pl.BlockSpec((1, D), lambda i: (0, 0)),
        pl.BlockSpec((D, D), lambda i: (0, 0)),
    ],
    out_specs=pl.BlockSpec((BM, D), lambda i: (i, 0)),
    out_shape=jax.ShapeDtypeStruct((N, D), jnp.float32),
)


def _tcfin_body(sp_ref, p_ref, dinv_ref, b_ref, out_ref):
    dcol = dinv_ref[:, 0:1]
    out_ref[...] = (sp_ref[0] + sp_ref[1] + p_ref[...]) * dcol + b_ref[...]


_tcfin_call = pl.pallas_call(
    _tcfin_body,
    grid=(N // BM,),
    in_specs=[
        pl.BlockSpec((NC, BM, D), lambda i: (0, i, 0)),
        pl.BlockSpec((BM, D), lambda i: (i, 0)),
        pl.BlockSpec((BM, DEGW), lambda i: (i, 0)),
        pl.BlockSpec((1, D), lambda i: (0, 0)),
    ],
    out_specs=pl.BlockSpec((BM, D), lambda i: (i, 0)),
    out_shape=jax.ShapeDtypeStruct((N, D), jnp.float32),
)


def kernel(x, edge_index, W1, b1, W2, b2, W3, b3):
    dst = edge_index[1]
    pk1 = (edge_index[0] << 14) | edge_index[1]
    degp = _deg_call(dst).reshape(NW, N).T
    p1, dinv16 = _tc1_call(x, W1, degp)
    sp1 = _agg_call(p1, pk1).reshape(NC, N, D)
    p2 = _tcmid_call(sp1, p1, dinv16, b1.reshape(1, D), W2)
    sp2 = _agg_call(p2, pk1).reshape(NC, N, D)
    p3 = _tcmid_call(sp2, p2, dinv16, b2.reshape(1, D), W3)
    sp3 = _agg_call(p3, pk1).reshape(NC, N, D)
    return _tcfin_call(sp3, p3, dinv16, b3.reshape(1, D))


# speedup vs baseline: 1.2738x; 1.0008x over previous
import functools

import jax
import jax.numpy as jnp
from jax import lax
from jax.experimental import pallas as pl
from jax.experimental.pallas import tpu as pltpu
from jax.experimental.pallas import tpu_sc as plsc

N = 10000
E = 320000
D = 128
NC = 2
NS = 16
NW = NC * NS
EPT = E // NW
K = 80
NCHUNK = EPT // K
RPT = N // NS
DEGW = 16
L = 16

_mesh = plsc.VectorSubcoreMesh(core_axis_name="c", subcore_axis_name="s")


def _fill(ref, rows, width, value):
    def body(i, _):
        r = i // (width // L)
        j = i % (width // L)
        ref[r, pl.ds(j * L, L)] = jnp.full((L,), value, jnp.float32)
        return 0

    lax.fori_loop(0, rows * (width // L), body, 0)


PROW = N // L


def _deg_body(dst_hbm, out_hbm, dv, pdeg):
    c = lax.axis_index("c")
    s = lax.axis_index("s")
    wid = c * NS + s
    pltpu.sync_copy(dst_hbm.at[pl.ds(wid * EPT, EPT)], dv)

    def z(i, _):
        pdeg[pl.ds(i * L, L)] = jnp.zeros((L,), jnp.float32)
        return 0

    lax.fori_loop(0, N // L, z, 0)
    ones = jnp.ones((L,), jnp.float32)

    def acc(i, _):
        idx = dv[pl.ds(i * L, L)]
        plsc.addupdate_scatter(pdeg, [idx], ones)
        return 0

    lax.fori_loop(0, EPT // L, acc, 0)
    pltpu.sync_copy(pdeg, out_hbm.at[c, s])


_deg_call = functools.partial(
    pl.kernel,
    out_type=jax.ShapeDtypeStruct((NC, NS, N), jnp.float32),
    mesh=_mesh,
    compiler_params=pltpu.CompilerParams(needs_layout_passes=False),
    scratch_types=[
        pltpu.VMEM((EPT,), jnp.int32),
        pltpu.VMEM((N,), jnp.float32),
    ],
)(_deg_body)


NB = 3


def _agg_body(p_hbm, pk_hbm, out_hbm, pks, sidx3, didx3, buf0, buf1, buf2,
              acc_sh, g0, g1, g2, i0, i1, i2, s0, s1, s2):
    c = lax.axis_index("c")
    s = lax.axis_index("s")
    wid = c * NS + s
    ebase = wid * EPT
    bufs = [buf0, buf1, buf2]
    gsems = [g0, g1, g2]
    isems = [i0, i1, i2]
    ssems = [s0, s1, s2]

    def fire_idx(q, j):
        pltpu.async_copy(pk_hbm.at[pl.ds(ebase + q * K, K)], pks.at[j],
                         isems[j])

    def wait_idx(j):
        pltpu.make_async_copy(pk_hbm.at[pl.ds(ebase, K)], pks.at[j],
                              isems[j]).wait()

    def unpack_s(j):
        for t in range(K // L):
            v = pks[j, pl.ds(t * L, L)]
            sidx3[j, pl.ds(t * L, L)] = lax.shift_right_logical(v, 14)

    def unpack_d(j):
        for t in range(K // L):
            v = pks[j, pl.ds(t * L, L)]
            didx3[j, pl.ds(t * L, L)] = lax.bitwise_and(v, 16383)

    def fire_g(j):
        pltpu.async_copy(p_hbm.at[sidx3.at[j]], bufs[j], gsems[j])

    def wait_g(j):
        pltpu.make_async_copy(p_hbm.at[sidx3.at[j]], bufs[j],
                              gsems[j]).wait()

    def fire_scat(j):
        pltpu.async_copy(bufs[j], acc_sh.at[didx3.at[j]], ssems[j],
                         add=True)

    def wait_scat(j):
        pltpu.make_async_copy(bufs[j], acc_sh.at[didx3.at[j]],
                              ssems[j]).wait()

    fire_idx(0, 0)
    fire_idx(1, 1)
    fire_idx(2, 2)

    _fill(buf0, K, D, 0.0)
    row0 = s * RPT
    for t in range(RPT // K):
        pltpu.sync_copy(buf0, acc_sh.at[pl.ds(row0 + t * K, K)])
    rem = RPT % K
    if rem:
        pltpu.sync_copy(buf0.at[pl.ds(0, rem)],
                        acc_sh.at[pl.ds(row0 + (RPT // K) * K, rem)])
    plsc.subcore_barrier()

    wait_idx(0)
    unpack_s(0)
    unpack_d(0)
    fire_g(0)
    wait_idx(1)
    unpack_s(1)
    unpack_d(1)
    fire_g(1)

    def step(i, _):
        for jj in range(NB):
            q = NB * i + jj
            jn = (jj + 2) % NB
            wait_idx(jn)
            unpack_s(jn)
            if jj == 0:
                @pl.when(i > 0)
                def _():
                    wait_scat(jn)
            else:
                wait_scat(jn)
            unpack_d(jn)
            fire_g(jn)
            wait_g(jj)
            fire_scat(jj)

            @pl.when(q + NB <= NCHUNK - 1)
            def _():
                fire_idx(q + NB, jj)

        return 0

    lax.fori_loop(0, (NCHUNK - 2) // NB, step, 0)
    wait_g((NCHUNK - 2) % NB)
    fire_scat((NCHUNK - 2) % NB)
    wait_g((NCHUNK - 1) % NB)
    fire_scat((NCHUNK - 1) % NB)
    for j in range(NB):
        wait_scat(j)
    plsc.subcore_barrier()
    pltpu.sync_copy(acc_sh.at[pl.ds(row0, RPT)], out_hbm.at[c, s])


_agg_call = functools.partial(
    pl.kernel,
    out_type=jax.ShapeDtypeStruct((NC, NS, RPT, D), jnp.float32),
    mesh=_mesh,
    scratch_types=[
        pltpu.VMEM((NB, K), jnp.int32),
        pltpu.VMEM((NB, K), jnp.int32),
        pltpu.VMEM((NB, K), jnp.int32),
        pltpu.VMEM((K, D), jnp.float32),
        pltpu.VMEM((K, D), jnp.float32),
        pltpu.VMEM((K, D), jnp.float32),
        pltpu.VMEM_SHARED((N, D), jnp.float32),
        pltpu.SemaphoreType.DMA,
        pltpu.SemaphoreType.DMA,
        pltpu.SemaphoreType.DMA,
        pltpu.SemaphoreType.DMA,
        pltpu.SemaphoreType.DMA,
        pltpu.SemaphoreType.DMA,
        pltpu.SemaphoreType.DMA,
        pltpu.SemaphoreType.DMA,
        pltpu.SemaphoreType.DMA,
    ],
)(_agg_body)


BM = 2000


def _tc1_body(x_ref, w_ref, degp_ref, p_ref, dinv_ref):
    deg = jnp.sum(degp_ref[...], axis=1, keepdims=True) + 1.0
    dcol = lax.rsqrt(deg)
    dinv_ref[...] = jnp.broadcast_to(dcol, (BM, DEGW))
    p_ref[...] = jnp.dot(x_ref[...], w_ref[...],
                         preferred_element_type=jnp.float32) * dcol


_tc1_call = pl.pallas_call(
    _tc1_body,
    grid=(N // BM,),
    in_specs=[
        pl.BlockSpec((BM, D), lambda i: (i, 0)),
        pl.BlockSpec((D, D), lambda i: (0, 0)),
        pl.BlockSpec((BM, NW), lambda i: (i, 0)),
    ],
    out_specs=[
        pl.BlockSpec((BM, D), lambda i: (i, 0)),
        pl.BlockSpec((BM, DEGW), lambda i: (i, 0)),
    ],
    out_shape=[
        jax.ShapeDtypeStruct((N, D), jnp.float32),
        jax.ShapeDtypeStruct((N, DEGW), jnp.float32),
    ],
)


def _tcmid_body(sp_ref, p_ref, dinv_ref, b_ref, w_ref, out_ref):
    dcol = dinv_ref[:, 0:1]
    t = (sp_ref[0] + sp_ref[1] + p_ref[...]) * dcol + b_ref[...]
    t = jnp.maximum(t, 0.0)
    out_ref[...] = jnp.dot(t, w_ref[...],
                           preferred_element_type=jnp.float32) * dcol


_tcmid_call = pl.pallas_call(
    _tcmid_body,
    grid=(N // BM,),
    in_specs=[
        pl.BlockSpec((NC, BM, D), lambda i: (0, i, 0)),
        pl.BlockSpec((BM, D), lambda i: (i, 0)),
        pl.BlockSpec((BM, DEGW), lambda i: (i, 0)),
        pl.BlockSpec((1, D), lambda i: (0, 0)),
        pl.BlockSpec((D, D), lambda i: (0, 0)),
    ],
    out_specs=pl.BlockSpec((BM, D), lambda i: (i, 0)),
    out_shape=jax.ShapeDtypeStruct((N, D), jnp.float32),
)


def _tcfin_body(sp_ref, p_ref, dinv_ref, b_ref, out_ref):
    dcol = dinv_ref[:, 0:1]
    out_ref[...] = (sp_ref[0] + sp_ref[1] + p_ref[...]) * dcol + b_ref[...]


_tcfin_call = pl.pallas_call(
    _tcfin_body,
    grid=(N // BM,),
    in_specs=[
        pl.BlockSpec((NC, BM, D), lambda i: (0, i, 0)),
        pl.BlockSpec((BM, D), lambda i: (i, 0)),
        pl.BlockSpec((BM, DEGW), lambda i: (i, 0)),
        pl.BlockSpec((1, D), lambda i: (0, 0)),
    ],
    out_specs=pl.BlockSpec((BM, D), lambda i: (i, 0)),
    out_shape=jax.ShapeDtypeStruct((N, D), jnp.float32),
)


def kernel(x, edge_index, W1, b1, W2, b2, W3, b3):
    dst = edge_index[1]
    pk1 = (edge_index[0] << 14) | edge_index[1]
    degp = _deg_call(dst).reshape(NW, N).T
    p1, dinv16 = _tc1_call(x, W1, degp)
    sp1 = _agg_call(p1, pk1).reshape(NC, N, D)
    p2 = _tcmid_call(sp1, p1, dinv16, b1.reshape(1, D), W2)
    sp2 = _agg_call(p2, pk1).reshape(NC, N, D)
    p3 = _tcmid_call(sp2, p2, dinv16, b2.reshape(1, D), W3)
    sp3 = _agg_call(p3, pk1).reshape(NC, N, D)
    return _tcfin_call(sp3, p3, dinv16, b3.reshape(1, D))
